# Initial kernel scaffold; baseline (speedup 1.0000x reference)
import jax
import jax.numpy as jnp
from jax.experimental import pallas as pl


def kernel(features, edge_weights, W1, W2, alpha1, gamma1, beta1, alpha2, gamma2, beta2, Wp1, bp1, Wr1, br1, Wp2, bp2, Wr2, br2, Wc, edge_index):
    raise NotImplementedError("write your pallas kernel here")


# speedup vs baseline: 2.2110x; 2.2110x over previous
import functools

import jax
import jax.numpy as jnp
from jax import lax
from jax.experimental import pallas as pl
from jax.experimental.pallas import tpu as pltpu
from jax.experimental.pallas import tpu_sc as plsc

N = 10000
E = 320000
D = 128
NC = 2
NS = 16
NW = NC * NS
EPW = E // NW
B = 80
NB = EPW // B
NP = 10240
CHK = 80
NCHK = NP // CHK
CPT = NCHK // NS
DEGW = 16
HN = 5120
HNP = 5440
NCHKM = HNP // CHK
CPTM = 5
NBT = E // NS // B
EPS = 1e-05


def _leaky(x):
    return jnp.where(x >= 0, x, 0.01 * x)


def _deg_body(src_hbm, dst_hbm, out_hbm, src_v, dst_v, ones_v, bnc_v, idx_b,
              acc):
    c = lax.axis_index("c")
    s = lax.axis_index("s")
    wid = s * NC + c

    def fill_ones(i, _):
        ones_v[i] = jnp.ones((DEGW,), jnp.float32)
        return 0

    lax.fori_loop(0, B, fill_ones, 0)

    def fill_zero(i, _):
        bnc_v[i] = jnp.zeros((DEGW,), jnp.float32)
        return 0

    lax.fori_loop(0, CHK, fill_zero, 0)

    pltpu.sync_copy(src_hbm.at[wid], src_v)
    pltpu.sync_copy(dst_hbm.at[wid], dst_v)

    for phase, idx_v in ((0, src_v), (1, dst_v)):
        for k in range(CPT):
            sl = pl.ds((s * CPT + k) * CHK, CHK)
            pltpu.sync_copy(bnc_v, acc.at[sl])

        plsc.subcore_barrier()

        def body(j, _):
            for q in range(B // 16):
                qsl = pl.ds(q * 16, 16)
                idx_b[qsl] = idx_v[j, qsl]
            pltpu.sync_copy(ones_v, acc.at[idx_b], add=True)
            return 0

        lax.fori_loop(0, NB, body, 0)
        plsc.subcore_barrier()

        for k in range(CPT):
            sl = pl.ds((s * CPT + k) * CHK, CHK)
            pltpu.sync_copy(acc.at[sl], bnc_v)
            pltpu.sync_copy(bnc_v, out_hbm.at[c, phase, sl])

        plsc.subcore_barrier()


def _msg_body(h_hbm, src_hbm, dst_hbm, ew_hbm, out_hbm,
              src_v, dst_v, ew_v, rows_v, bnc_v, idx_b, sem, acc):
    c = lax.axis_index("c")
    s = lax.axis_index("s")

    def fill_zero(i, _):
        for kk in range(D // 16):
            bnc_v[i, pl.ds(kk * 16, 16)] = jnp.zeros((16,), jnp.float32)
        return 0

    lax.fori_loop(0, CHK, fill_zero, 0)

    for k in range(CPTM):
        cid = s * CPTM + k

        @pl.when(cid < NCHKM)
        def _():
            pltpu.sync_copy(bnc_v, acc.at[pl.ds(cid * CHK, CHK)])

    plsc.subcore_barrier()

    pltpu.sync_copy(src_hbm.at[s], src_v)
    pltpu.sync_copy(dst_hbm.at[s], dst_v)

    lov = jnp.full((16,), c * HN, jnp.int32)
    szv = jnp.full((16,), HN - c * (2 * HN - N), jnp.int32)
    trv = jnp.full((16,), HN + s, jnp.int32)
    zv = jnp.zeros((16,), jnp.int32)

    def remap(r, _):
        for q in range(B // 16):
            sl = pl.ds(q * 16, 16)
            t = dst_v[r, sl] - lov
            m = (t >= zv) & (t < szv)
            dst_v[r, sl] = jnp.where(m, t, trv)
        return 0

    lax.fori_loop(0, NBT, remap, 0)

    def body(j, _):
        pltpu.sync_copy(ew_hbm.at[s, j], ew_v)
        pltpu.async_copy(h_hbm.at[src_v.at[j]], rows_v, sem).wait()

        def scale(g, _):
            ws = ew_v[pl.ds(g * 16, 16)]
            base = g * 16
            for l in range(16):
                w = jnp.full((16,), ws[l], jnp.float32)
                for kk in range(D // 16):
                    sl = pl.ds(kk * 16, 16)
                    rows_v[base + l, sl] = rows_v[base + l, sl] * w
            return 0

        lax.fori_loop(0, B // 16, scale, 0)
        for q in range(B // 16):
            qsl = pl.ds(q * 16, 16)
            idx_b[qsl] = dst_v[j, qsl]
        pltpu.sync_copy(rows_v, acc.at[idx_b], add=True)
        return 0

    lax.fori_loop(0, NBT, body, 0)
    plsc.subcore_barrier()

    for k in range(CPTM):
        cid = s * CPTM + k

        @pl.when(cid < NCHKM)
        def _():
            sl = pl.ds(cid * CHK, CHK)
            pltpu.sync_copy(acc.at[sl], bnc_v)
            pltpu.sync_copy(bnc_v, out_hbm.at[c, sl])


@functools.cache
def _deg_kernel():
    mesh = plsc.VectorSubcoreMesh(
        core_axis_name="c", subcore_axis_name="s",
        num_cores=NC, num_subcores=NS)
    return pl.kernel(
        _deg_body,
        out_type=jax.ShapeDtypeStruct((NC, 2, NP, DEGW), jnp.float32),
        mesh=mesh,
        scratch_types=[
            pltpu.VMEM((NB, B), jnp.int32),
            pltpu.VMEM((NB, B), jnp.int32),
            pltpu.VMEM((B, DEGW), jnp.float32),
            pltpu.VMEM((CHK, DEGW), jnp.float32),
            pltpu.VMEM((B,), jnp.int32),
            pltpu.VMEM_SHARED((NP, DEGW), jnp.float32),
        ],
    )


@functools.cache
def _msg_kernel():
    mesh = plsc.VectorSubcoreMesh(
        core_axis_name="c", subcore_axis_name="s",
        num_cores=NC, num_subcores=NS)
    return pl.kernel(
        _msg_body,
        out_type=jax.ShapeDtypeStruct((NC, HNP, D), jnp.float32),
        mesh=mesh,
        scratch_types=[
            pltpu.VMEM((NBT, B), jnp.int32),
            pltpu.VMEM((NBT, B), jnp.int32),
            pltpu.VMEM((B,), jnp.float32),
            pltpu.VMEM((B, D), jnp.float32),
            pltpu.VMEM((CHK, D), jnp.float32),
            pltpu.VMEM((B,), jnp.int32),
            pltpu.SemaphoreType.DMA,
            pltpu.VMEM_SHARED((HNP, D), jnp.float32),
        ],
    )


def _gprobe_body(h_hbm, idx_hbm, out_hbm, idx_v, rows_v, bnc_v, sem, sh):
    c = lax.axis_index("c")
    s = lax.axis_index("s")
    wid = s * NC + c
    pltpu.sync_copy(idx_hbm.at[wid, 0], idx_v)
    for q in range(B // 16):
        qsl = pl.ds(q * 16, 16)
        idx_v[qsl] = idx_v[qsl] & 1023
    pltpu.async_copy(h_hbm.at[idx_v], rows_v, sem).wait()

    def fz(i, _):
        for kk in range(D // 16):
            bnc_v[i, pl.ds(kk * 16, 16)] = jnp.zeros((16,), jnp.float32)
        return 0

    lax.fori_loop(0, B, fz, 0)
    sl = pl.ds(s * B, B)
    pltpu.sync_copy(bnc_v, sh.at[sl])
    plsc.subcore_barrier()
    pltpu.sync_copy(rows_v, sh.at[idx_v], add=True)
    plsc.subcore_barrier()
    pltpu.sync_copy(sh.at[sl], rows_v)
    pltpu.sync_copy(rows_v, out_hbm.at[wid])


@functools.cache
def _gprobe_kernel():
    mesh = plsc.VectorSubcoreMesh(
        core_axis_name="c", subcore_axis_name="s",
        num_cores=NC, num_subcores=NS)
    return pl.kernel(
        _gprobe_body,
        out_type=jax.ShapeDtypeStruct((NW, B, D), jnp.float32),
        mesh=mesh,
        scratch_types=[
            pltpu.VMEM((B,), jnp.int32),
            pltpu.VMEM((B, D), jnp.float32),
            pltpu.VMEM((B, D), jnp.float32),
            pltpu.SemaphoreType.DMA,
            pltpu.VMEM_SHARED((NS * B, D), jnp.float32),
        ],
    )


def _norm_col(p0, p1):
    deg = p0[:, 0:1] + p1[:, 0:1]
    return jnp.where(deg > 0, lax.rsqrt(deg), 0.0)


def _tcA_body(do0_ref, do1_ref, x_ref, w1_ref, h_ref):
    ns = _norm_col(do0_ref[...], do1_ref[...])
    xn = x_ref[...] * ns
    h_ref[...] = lax.dot_general(
        xn, w1_ref[...], (((1,), (1,)), ((), ())),
        preferred_element_type=jnp.float32)


def _graphnorm_leaky(agg, alpha, gamma, beta):
    mean = jnp.mean(agg, axis=0, keepdims=True)
    sub = agg - alpha * mean
    var = jnp.mean(sub * sub, axis=0, keepdims=True)
    return _leaky(gamma * sub * lax.rsqrt(var + EPS) + beta)


def _readout(h, wp_ref, bp_ref, wr_ref, br_ref):
    ssum = jnp.zeros((1, 2 * D), jnp.float32)
    nchunk = 10
    rows = N // nchunk
    for i in range(nchunk):
        hc = h[i * rows:(i + 1) * rows]
        phi = _leaky(lax.dot_general(
            hc, wp_ref[...], (((1,), (1,)), ((), ())),
            preferred_element_type=jnp.float32) + bp_ref[...])
        ssum = ssum + jnp.sum(phi, axis=0, keepdims=True)
    s = ssum / N
    return _leaky(lax.dot_general(
        s, wr_ref[...], (((1,), (1,)), ((), ())),
        preferred_element_type=jnp.float32) + br_ref[...])


def _combine(pa, pb):
    return jnp.concatenate([pa, pb], axis=0)


def _tcB_body(pa_ref, pb_ref, do0_ref, do1_ref, di0_ref, di1_ref,
              a1_ref, g1_ref, b1_ref, wp1_ref, bp1_ref, wr1_ref, br1_ref,
              w2_ref, h2_ref, r1_ref):
    nd = _norm_col(di0_ref[...], di1_ref[...])
    agg = _combine(pa_ref[...], pb_ref[...]) * nd
    h1 = _graphnorm_leaky(agg, a1_ref[...], g1_ref[...], b1_ref[...])
    r1_ref[...] = _readout(h1, wp1_ref, bp1_ref, wr1_ref, br1_ref)
    ns = _norm_col(do0_ref[...], do1_ref[...])
    h2_ref[...] = lax.dot_general(
        h1 * ns, w2_ref[...], (((1,), (1,)), ((), ())),
        preferred_element_type=jnp.float32)


def _tcC_body(pa_ref, pb_ref, di0_ref, di1_ref,
              a2_ref, g2_ref, b2_ref,
              wp2_ref, bp2_ref, wr2_ref, br2_ref, r1_ref, wc_ref, out_ref):
    nd = _norm_col(di0_ref[...], di1_ref[...])
    agg = _combine(pa_ref[...], pb_ref[...]) * nd
    h2 = _graphnorm_leaky(agg, a2_ref[...], g2_ref[...], b2_ref[...])
    r2 = _readout(h2, wp2_ref, bp2_ref, wr2_ref, br2_ref)
    ro = jnp.concatenate([r1_ref[...], r2], axis=1)
    out_ref[...] = lax.dot_general(
        ro, wc_ref[...], (((1,), (1,)), ((), ())),
        preferred_element_type=jnp.float32)


def _tc_call(body, out_shape, *args):
    return pl.pallas_call(body, out_shape=out_shape)(*args)


def kernel(features, edge_weights, W1, W2, alpha1, gamma1, beta1,
           alpha2, gamma2, beta2, Wp1, bp1, Wr1, br1, Wp2, bp2, Wr2, br2,
           Wc, edge_index):
    srcw = edge_index[0].reshape(NW, NB, B)
    dstw = edge_index[1].reshape(NW, NB, B)
    srct = edge_index[0].reshape(NS, NBT, B)
    dstt = edge_index[1].reshape(NS, NBT, B)
    ewt = edge_weights.reshape(NS, NBT, B)

    do_f = jnp.zeros((N,), jnp.float32).at[edge_index[0]].add(1.0)
    di_f = jnp.zeros((N,), jnp.float32).at[edge_index[1]].add(1.0)
    do0 = jnp.broadcast_to(do_f[:, None], (N, DEGW))
    do1 = jnp.zeros((N, DEGW), jnp.float32)
    di0 = jnp.broadcast_to(di_f[:, None], (N, DEGW))
    di1 = jnp.zeros((N, DEGW), jnp.float32)

    h1pre = _tc_call(_tcA_body, jax.ShapeDtypeStruct((N, D), jnp.float32),
                     do0, do1, features, W1)

    msg = _msg_kernel()
    p1 = msg(h1pre, srct, dstt, ewt)

    h2pre, r1 = _tc_call(
        _tcB_body,
        [jax.ShapeDtypeStruct((N, D), jnp.float32),
         jax.ShapeDtypeStruct((1, D // 4), jnp.float32)],
        p1[0, :HN], p1[1, :N - HN],
        do0, do1, di0, di1,
        alpha1.reshape(1, D), gamma1.reshape(1, D), beta1.reshape(1, D),
        Wp1, bp1.reshape(1, 2 * D), Wr1, br1.reshape(1, D // 4), W2)

    p2 = msg(h2pre, srct, dstt, ewt)

    out = _tc_call(
        _tcC_body, jax.ShapeDtypeStruct((1, 10), jnp.float32),
        p2[0, :HN], p2[1, :N - HN], di0, di1,
        alpha2.reshape(1, D), gamma2.reshape(1, D), beta2.reshape(1, D),
        Wp2, bp2.reshape(1, 2 * D), Wr2, br2.reshape(1, D // 4), r1, Wc)
    return out
